# Initial kernel scaffold; baseline (speedup 1.0000x reference)
import jax
import jax.numpy as jnp
from jax.experimental import pallas as pl


def kernel(x_node_features, batch_vector):
    raise NotImplementedError("write your pallas kernel here")


# speedup vs baseline: 3.6209x; 3.6209x over previous
import jax
import jax.numpy as jnp
from jax import lax
from jax.experimental import pallas as pl
from jax.experimental.pallas import tpu as pltpu
from jax.experimental.pallas import tpu_sc as plsc
import functools

N = 100000
D = 128
S = 64
NS = 16
ROWS_PER_W = N // NS
CHUNK = 125
CHUNKS_PER_W = ROWS_PER_W // CHUNK
SEGS_PER_W = S // NS
LANES = 16

_mesh = plsc.VectorSubcoreMesh(
    core_axis_name="c", subcore_axis_name="s", num_cores=1, num_subcores=NS)


@functools.partial(
    pl.kernel,
    out_type=jax.ShapeDtypeStruct((S, D), jnp.float32),
    mesh=_mesh,
    scratch_types=[
        pltpu.VMEM((CHUNKS_PER_W, CHUNK), jnp.int32),
        pltpu.VMEM((CHUNK, D), jnp.float32),
        pltpu.VMEM((CHUNK, D), jnp.float32),
        pltpu.VMEM((CHUNK, LANES), jnp.float32),
        pltpu.VMEM((SEGS_PER_W, D), jnp.float32),
        pltpu.VMEM((SEGS_PER_W, LANES), jnp.float32),
        pltpu.VMEM_SHARED((S, D), jnp.float32),
        pltpu.VMEM_SHARED((S, LANES), jnp.float32),
        pltpu.SemaphoreType.DMA,
        pltpu.SemaphoreType.DMA,
    ],
    compiler_params=pltpu.CompilerParams(use_tc_tiling_on_sc=False),
)
def _pool_sc(x_hbm, idx_hbm, out_hbm,
             idx_v, xbuf0, xbuf1, ones_v, sbuf, cbuf,
             shared_sum, shared_cnt, sem0, sem1):
    sid = lax.axis_index("s")

    zeros16 = jnp.zeros((LANES,), jnp.float32)
    ones16 = jnp.ones((LANES,), jnp.float32)

    for r in range(SEGS_PER_W):
        for g in range(D // LANES):
            sbuf[r, pl.ds(g * LANES, LANES)] = zeros16
        cbuf[r, :] = zeros16
    pltpu.sync_copy(sbuf, shared_sum.at[pl.ds(sid * SEGS_PER_W, SEGS_PER_W)])
    pltpu.sync_copy(cbuf, shared_cnt.at[pl.ds(sid * SEGS_PER_W, SEGS_PER_W)])

    for r in range(CHUNK):
        ones_v[r, :] = ones16

    pltpu.sync_copy(idx_hbm.at[pl.ds(sid * CHUNKS_PER_W, CHUNKS_PER_W)], idx_v)

    plsc.subcore_barrier()

    xbufs = (xbuf0, xbuf1)
    sems = (sem0, sem1)
    row0 = sid * ROWS_PER_W
    copies = [None, None]

    copies[0] = pltpu.async_copy(x_hbm.at[pl.ds(row0, CHUNK)], xbuf0, sem0)
    for k in range(CHUNKS_PER_W):
        if k + 1 < CHUNKS_PER_W:
            b = (k + 1) % 2
            copies[b] = pltpu.async_copy(
                x_hbm.at[pl.ds(row0 + (k + 1) * CHUNK, CHUNK)], xbufs[b],
                sems[b])
        copies[k % 2].wait()
        idx_row = idx_v.at[k]
        pltpu.sync_copy(xbufs[k % 2], shared_sum.at[idx_row], add=True)
        pltpu.sync_copy(ones_v, shared_cnt.at[idx_row], add=True)

    plsc.subcore_barrier()

    seg0 = sid * SEGS_PER_W
    pltpu.sync_copy(shared_sum.at[pl.ds(seg0, SEGS_PER_W)], sbuf)
    pltpu.sync_copy(shared_cnt.at[pl.ds(seg0, SEGS_PER_W)], cbuf)
    for r in range(SEGS_PER_W):
        cnt = jnp.maximum(cbuf[r, :], 1.0)
        for g in range(D // LANES):
            sl = pl.ds(g * LANES, LANES)
            sbuf[r, sl] = sbuf[r, sl] / cnt
    pltpu.sync_copy(sbuf, out_hbm.at[pl.ds(seg0, SEGS_PER_W)])


def kernel(x_node_features, batch_vector):
    idx2d = batch_vector.astype(jnp.int32).reshape(N // CHUNK, CHUNK)
    return _pool_sc(x_node_features, idx2d)


# speedup vs baseline: 5.4253x; 1.4983x over previous
import jax
import jax.numpy as jnp
from jax import lax
from jax.experimental import pallas as pl
from jax.experimental.pallas import tpu as pltpu
from jax.experimental.pallas import tpu_sc as plsc
import functools

N = 100000
D = 128
S = 64
NC = 2
NS = 16
DCOL = D // NC
ROWS_PER_W = N // NS
CHUNK = 125
CHUNKS_PER_W = ROWS_PER_W // CHUNK
SEGS_PER_W = S // NS
LANES = 16

_mesh = plsc.VectorSubcoreMesh(
    core_axis_name="c", subcore_axis_name="s", num_cores=NC, num_subcores=NS)


@functools.partial(
    pl.kernel,
    out_type=jax.ShapeDtypeStruct((S, D), jnp.float32),
    mesh=_mesh,
    scratch_types=[
        pltpu.VMEM((CHUNKS_PER_W, CHUNK), jnp.int32),
        pltpu.VMEM((CHUNK, DCOL), jnp.float32),
        pltpu.VMEM((CHUNK, DCOL), jnp.float32),
        pltpu.VMEM((CHUNK, LANES), jnp.float32),
        pltpu.VMEM((SEGS_PER_W, DCOL), jnp.float32),
        pltpu.VMEM((SEGS_PER_W, LANES), jnp.float32),
        pltpu.VMEM_SHARED((S, DCOL), jnp.float32),
        pltpu.VMEM_SHARED((S, LANES), jnp.float32),
        pltpu.SemaphoreType.DMA,
        pltpu.SemaphoreType.DMA,
    ],
    compiler_params=pltpu.CompilerParams(use_tc_tiling_on_sc=False),
)
def _pool_sc(x_hbm, idx_hbm, out_hbm,
             idx_v, xbuf0, xbuf1, ones_v, sbuf, cbuf,
             shared_sum, shared_cnt, sem0, sem1):
    cid = lax.axis_index("c")
    sid = lax.axis_index("s")
    col0 = cid * DCOL

    zeros16 = jnp.zeros((LANES,), jnp.float32)
    ones16 = jnp.ones((LANES,), jnp.float32)

    for r in range(SEGS_PER_W):
        for g in range(DCOL // LANES):
            sbuf[r, pl.ds(g * LANES, LANES)] = zeros16
        cbuf[r, :] = zeros16
    pltpu.sync_copy(sbuf, shared_sum.at[pl.ds(sid * SEGS_PER_W, SEGS_PER_W)])
    pltpu.sync_copy(cbuf, shared_cnt.at[pl.ds(sid * SEGS_PER_W, SEGS_PER_W)])

    for r in range(CHUNK):
        ones_v[r, :] = ones16

    pltpu.sync_copy(idx_hbm.at[pl.ds(sid * CHUNKS_PER_W, CHUNKS_PER_W)], idx_v)

    plsc.subcore_barrier()

    xbufs = (xbuf0, xbuf1)
    sems = (sem0, sem1)
    row0 = sid * ROWS_PER_W
    copies = [None, None]

    copies[0] = pltpu.async_copy(
        x_hbm.at[pl.ds(row0, CHUNK), pl.ds(col0, DCOL)], xbuf0, sem0)
    for k in range(CHUNKS_PER_W):
        if k + 1 < CHUNKS_PER_W:
            b = (k + 1) % 2
            copies[b] = pltpu.async_copy(
                x_hbm.at[pl.ds(row0 + (k + 1) * CHUNK, CHUNK),
                         pl.ds(col0, DCOL)],
                xbufs[b], sems[b])
        copies[k % 2].wait()
        idx_row = idx_v.at[k]
        pltpu.sync_copy(xbufs[k % 2], shared_sum.at[idx_row], add=True)
        pltpu.sync_copy(ones_v, shared_cnt.at[idx_row], add=True)

    plsc.subcore_barrier()

    seg0 = sid * SEGS_PER_W
    pltpu.sync_copy(shared_sum.at[pl.ds(seg0, SEGS_PER_W)], sbuf)
    pltpu.sync_copy(shared_cnt.at[pl.ds(seg0, SEGS_PER_W)], cbuf)
    for r in range(SEGS_PER_W):
        cnt = jnp.maximum(cbuf[r, :], 1.0)
        for g in range(DCOL // LANES):
            sl = pl.ds(g * LANES, LANES)
            sbuf[r, sl] = sbuf[r, sl] / cnt
    pltpu.sync_copy(sbuf, out_hbm.at[pl.ds(seg0, SEGS_PER_W), pl.ds(col0, DCOL)])


def kernel(x_node_features, batch_vector):
    idx2d = batch_vector.astype(jnp.int32).reshape(N // CHUNK, CHUNK)
    return _pool_sc(x_node_features, idx2d)


# speedup vs baseline: 7.5119x; 1.3846x over previous
import jax
import jax.numpy as jnp
from jax import lax
from jax.experimental import pallas as pl
from jax.experimental.pallas import tpu as pltpu
from jax.experimental.pallas import tpu_sc as plsc
import functools

N = 100000
D = 128
S = 64
NC = 2
NS = 16
DCOL = D // NC
ROWS_PER_W = N // NS
CHUNK = 125
CHUNKS_PER_W = ROWS_PER_W // CHUNK
SEGS_PER_W = S // NS
LANES = 16
NG = DCOL // LANES

_mesh = plsc.VectorSubcoreMesh(
    core_axis_name="c", subcore_axis_name="s", num_cores=NC, num_subcores=NS)


@functools.partial(
    pl.kernel,
    out_type=jax.ShapeDtypeStruct((S, D), jnp.float32),
    mesh=_mesh,
    scratch_types=[
        pltpu.VMEM((CHUNKS_PER_W, CHUNK), jnp.int32),
        pltpu.VMEM((CHUNK, DCOL), jnp.float32),
        pltpu.VMEM((CHUNK, DCOL), jnp.float32),
        pltpu.VMEM((CHUNK, LANES), jnp.float32),
        pltpu.VMEM((S, DCOL), jnp.float32),
        pltpu.VMEM((S, LANES), jnp.float32),
        pltpu.VMEM((1, S), jnp.int32),
        pltpu.VMEM((SEGS_PER_W, DCOL), jnp.float32),
        pltpu.VMEM((SEGS_PER_W, LANES), jnp.float32),
        pltpu.VMEM_SHARED((S, DCOL), jnp.float32),
        pltpu.VMEM_SHARED((S, LANES), jnp.float32),
        pltpu.SemaphoreType.DMA,
        pltpu.SemaphoreType.DMA,
    ],
    compiler_params=pltpu.CompilerParams(use_tc_tiling_on_sc=False),
)
def _pool_sc(x_hbm, idx_hbm, out_hbm,
             idx_v, xbuf0, xbuf1, ones_v, acc_local, cnt_local, idbuf,
             sbuf, cbuf, shared_sum, shared_cnt, sem0, sem1):
    cid = lax.axis_index("c")
    sid = lax.axis_index("s")
    col0 = cid * DCOL

    zeros16 = jnp.zeros((LANES,), jnp.float32)
    ones16 = jnp.ones((LANES,), jnp.float32)

    for r in range(SEGS_PER_W):
        for g in range(NG):
            sbuf[r, pl.ds(g * LANES, LANES)] = zeros16
        cbuf[r, :] = zeros16
    pltpu.sync_copy(sbuf, shared_sum.at[pl.ds(sid * SEGS_PER_W, SEGS_PER_W)])
    pltpu.sync_copy(cbuf, shared_cnt.at[pl.ds(sid * SEGS_PER_W, SEGS_PER_W)])

    for r in range(CHUNK):
        ones_v[r, :] = ones16

    for g in range(S // LANES):
        idbuf[0, pl.ds(g * LANES, LANES)] = (
            lax.iota(jnp.int32, LANES) + g * LANES)

    def _zero_body(r, _):
        for g in range(NG):
            acc_local[r, pl.ds(g * LANES, LANES)] = zeros16
        cnt_local[r, :] = zeros16
        return 0
    lax.fori_loop(0, S, _zero_body, 0)

    pltpu.sync_copy(idx_hbm.at[pl.ds(sid * CHUNKS_PER_W, CHUNKS_PER_W)], idx_v)

    plsc.subcore_barrier()

    xbufs = (xbuf0, xbuf1)
    sems = (sem0, sem1)
    row0 = sid * ROWS_PER_W
    copies = [None, None]

    copies[0] = pltpu.async_copy(
        x_hbm.at[pl.ds(row0, CHUNK), pl.ds(col0, DCOL)], xbuf0, sem0)
    for k in range(CHUNKS_PER_W):
        if k + 1 < CHUNKS_PER_W:
            b = (k + 1) % 2
            copies[b] = pltpu.async_copy(
                x_hbm.at[pl.ds(row0 + (k + 1) * CHUNK, CHUNK),
                         pl.ds(col0, DCOL)],
                xbufs[b], sems[b])
        copies[k % 2].wait()
        xb = xbufs[k % 2]
        idx_row = idx_v.at[k]

        seg_lo = idx_v[k, pl.ds(0, LANES)][0]
        seg_hi = idx_v[k, pl.ds(CHUNK - LANES, LANES)][LANES - 1]
        pure = seg_lo == seg_hi

        @pl.when(pure)
        def _pure():
            def _body(r, acc):
                return tuple(
                    acc[g] + xb[r, pl.ds(g * LANES, LANES)]
                    for g in range(NG))
            acc = lax.fori_loop(
                0, CHUNK, _body, tuple(zeros16 for _ in range(NG)))
            for g in range(NG):
                sl = pl.ds(g * LANES, LANES)
                acc_local[seg_lo, sl] = acc_local[seg_lo, sl] + acc[g]
            cnt_local[seg_lo, :] = cnt_local[seg_lo, :] + float(CHUNK)

        @pl.when(jnp.logical_not(pure))
        def _impure():
            pltpu.sync_copy(xb, shared_sum.at[idx_row], add=True)
            pltpu.sync_copy(ones_v, shared_cnt.at[idx_row], add=True)

    id_row = idbuf.at[0]
    pltpu.sync_copy(acc_local, shared_sum.at[id_row], add=True)
    pltpu.sync_copy(cnt_local, shared_cnt.at[id_row], add=True)

    plsc.subcore_barrier()

    seg0 = sid * SEGS_PER_W
    pltpu.sync_copy(shared_sum.at[pl.ds(seg0, SEGS_PER_W)], sbuf)
    pltpu.sync_copy(shared_cnt.at[pl.ds(seg0, SEGS_PER_W)], cbuf)
    for r in range(SEGS_PER_W):
        cnt = jnp.maximum(cbuf[r, :], 1.0)
        for g in range(NG):
            sl = pl.ds(g * LANES, LANES)
            sbuf[r, sl] = sbuf[r, sl] / cnt
    pltpu.sync_copy(sbuf, out_hbm.at[pl.ds(seg0, SEGS_PER_W), pl.ds(col0, DCOL)])


def kernel(x_node_features, batch_vector):
    idx2d = batch_vector.astype(jnp.int32).reshape(N // CHUNK, CHUNK)
    return _pool_sc(x_node_features, idx2d)
